# baseline (speedup 1.0000x reference)
import functools

import jax
import jax.numpy as jnp
from jax.experimental import pallas as pl
from jax.experimental.pallas import tpu as pltpu

H = 1024
FF = 2048
E = 8
T = 512
TM = 128
NT = 16
NS = NT * TM
TB = T // TM


def _router_kernel(x_ref, rw_ref, pw_ref, counts_ref):
    x = x_ref[:, :]
    rw = rw_ref[:, :]
    logits = jax.lax.dot_general(
        x, rw, (((1,), (1,)), ((), ())), preferred_element_type=jnp.float32)
    m = jnp.max(logits, axis=1, keepdims=True)
    ex = jnp.exp(logits - m)
    probs = ex / jnp.sum(ex, axis=1, keepdims=True)

    ecols = jax.lax.broadcasted_iota(jnp.int32, (T, E), 1)
    m1 = jnp.max(probs, axis=1, keepdims=True)
    i1 = jnp.min(jnp.where(probs == m1, ecols, E), axis=1, keepdims=True)
    probs2 = jnp.where(ecols == i1, -jnp.inf, probs)
    m2 = jnp.max(probs2, axis=1, keepdims=True)
    i2 = jnp.min(jnp.where(probs2 == m2, ecols, E), axis=1, keepdims=True)
    s = m1 + m2
    w1 = m1 / s
    w2 = m2 / s

    sel1 = (ecols == i1)
    sel2 = (ecols == i2)
    memb = (sel1 | sel2).astype(jnp.float32)

    ri = jax.lax.broadcasted_iota(jnp.int32, (T, T), 0)
    ci = jax.lax.broadcasted_iota(jnp.int32, (T, T), 1)
    ltri = (ci < ri).astype(jnp.float32)
    rank = jax.lax.dot_general(
        ltri, memb, (((1,), (0,)), ((), ())), preferred_element_type=jnp.float32)

    counts = jnp.sum(memb, axis=0, keepdims=True)
    tiles = jnp.ceil(counts / TM)
    er = jax.lax.broadcasted_iota(jnp.int32, (E, E), 0)
    ec = jax.lax.broadcasted_iota(jnp.int32, (E, E), 1)
    ltri_e = (er < ec).astype(jnp.float32)
    base = jax.lax.dot_general(
        tiles, ltri_e, (((1,), (0,)), ((), ())),
        preferred_element_type=jnp.float32) * TM

    pos = base + rank
    p1 = jnp.sum(jnp.where(sel1, pos, 0.0), axis=1, keepdims=True)
    p2 = jnp.sum(jnp.where(sel2, pos, 0.0), axis=1, keepdims=True)

    pw_ref[:, :] = (jnp.where(ecols == 0, p1, 0.0) +
                    jnp.where(ecols == 1, p2, 0.0) +
                    jnp.where(ecols == 2, w1, 0.0) +
                    jnp.where(ecols == 3, w2, 0.0))
    counts_ref[:, :] = jnp.broadcast_to(counts, (E, E))


def _ffn_kernel(te_ref, ta_ref, x_ref, pt_ref, eg_ref, eu_ref, ed_ref, ys_ref):
    nt = pl.program_id(0)

    @pl.when(ta_ref[nt] == 1)
    def _active():
        si = (jax.lax.broadcasted_iota(jnp.int32, (TM, T), 0)
              + nt * TM).astype(jnp.float32)
        d = ((pt_ref[0:1, :] == si) | (pt_ref[1:2, :] == si)).astype(jnp.bfloat16)
        xs = jax.lax.dot_general(
            d, x_ref[:, :], (((1,), (0,)), ((), ())),
            preferred_element_type=jnp.float32).astype(jnp.bfloat16)
        xg = jax.lax.dot_general(
            xs, eg_ref[0], (((1,), (1,)), ((), ())),
            preferred_element_type=jnp.float32)
        xu = jax.lax.dot_general(
            xs, eu_ref[0], (((1,), (1,)), ((), ())),
            preferred_element_type=jnp.float32)
        h = (xg * jax.nn.sigmoid(xg) * xu).astype(jnp.bfloat16)
        ys_ref[:, :] = jax.lax.dot_general(
            h, ed_ref[0], (((1,), (1,)), ((), ())),
            preferred_element_type=jnp.float32)

    @pl.when(ta_ref[nt] == 0)
    def _inactive():
        ys_ref[:, :] = jnp.zeros((TM, H), jnp.float32)


def _combine_kernel(x_ref, pw_ref, ys_ref, wg_ref, wu_ref, wd_ref, o_ref):
    xb = x_ref[:, :]
    xg = jax.lax.dot_general(
        xb, wg_ref[:, :], (((1,), (1,)), ((), ())),
        preferred_element_type=jnp.float32)
    xu = jax.lax.dot_general(
        xb, wu_ref[:, :], (((1,), (1,)), ((), ())),
        preferred_element_type=jnp.float32)
    h = (xg * jax.nn.sigmoid(xg) * xu).astype(jnp.bfloat16)
    shared = jax.lax.dot_general(
        h, wd_ref[:, :], (((1,), (1,)), ((), ())),
        preferred_element_type=jnp.float32)

    p1 = pw_ref[:, 0:1]
    p2 = pw_ref[:, 1:2]
    w1 = pw_ref[:, 2:3]
    w2 = pw_ref[:, 3:4]
    sj = jax.lax.broadcasted_iota(jnp.int32, (TM, NS), 1).astype(jnp.float32)
    c = (w1 * (p1 == sj).astype(jnp.float32) +
         w2 * (p2 == sj).astype(jnp.float32)).astype(jnp.bfloat16)
    moe = jax.lax.dot_general(
        c, ys_ref[:, :], (((1,), (0,)), ((), ())),
        preferred_element_type=jnp.float32)
    o_ref[:, :] = shared + moe


@jax.jit
def kernel(x, router_weight, w_gate, w_up, w_down,
           expert_gate, expert_up, expert_down):
    f32 = jnp.float32

    pw, counts8 = pl.pallas_call(
        _router_kernel,
        out_shape=(jax.ShapeDtypeStruct((T, E), f32),
                   jax.ShapeDtypeStruct((E, E), f32)),
    )(x, router_weight)

    counts = counts8[0].astype(jnp.int32)
    tiles = (counts + TM - 1) // TM
    ends = jnp.cumsum(tiles)
    total = ends[E - 1]
    ntv = jnp.arange(NT, dtype=jnp.int32)
    ntc = jnp.minimum(ntv, total - 1)
    tile_expert = jnp.sum((ends[None, :] <= ntc[:, None]).astype(jnp.int32),
                          axis=1)
    tile_active = (ntv < total).astype(jnp.int32)

    pt = jnp.full((8, T), -1.0, f32)
    pt = pt.at[0].set(pw[:, 0]).at[1].set(pw[:, 1])

    x_bf = x.astype(jnp.bfloat16)
    eg_bf = expert_gate.astype(jnp.bfloat16)
    eu_bf = expert_up.astype(jnp.bfloat16)
    ed_bf = expert_down.astype(jnp.bfloat16)

    ys = pl.pallas_call(
        _ffn_kernel,
        grid_spec=pltpu.PrefetchScalarGridSpec(
            num_scalar_prefetch=2,
            grid=(NT,),
            in_specs=[
                pl.BlockSpec((T, H), lambda nt, te, ta: (0, 0)),
                pl.BlockSpec((8, T), lambda nt, te, ta: (0, 0)),
                pl.BlockSpec((1, FF, H), lambda nt, te, ta: (te[nt], 0, 0)),
                pl.BlockSpec((1, FF, H), lambda nt, te, ta: (te[nt], 0, 0)),
                pl.BlockSpec((1, H, FF), lambda nt, te, ta: (te[nt], 0, 0)),
            ],
            out_specs=pl.BlockSpec((TM, H), lambda nt, te, ta: (nt, 0)),
        ),
        out_shape=jax.ShapeDtypeStruct((NS, H), f32),
        compiler_params=pltpu.CompilerParams(
            dimension_semantics=("arbitrary",)),
    )(tile_expert, tile_active, x_bf, pt, eg_bf, eu_bf, ed_bf)

    ys_bf = ys.astype(jnp.bfloat16)
    wg_bf = w_gate.astype(jnp.bfloat16)
    wu_bf = w_up.astype(jnp.bfloat16)
    wd_bf = w_down.astype(jnp.bfloat16)

    out = pl.pallas_call(
        _combine_kernel,
        grid_spec=pl.GridSpec(
            grid=(TB,),
            in_specs=[
                pl.BlockSpec((TM, H), lambda tb: (tb, 0)),
                pl.BlockSpec((TM, E), lambda tb: (tb, 0)),
                pl.BlockSpec((NS, H), lambda tb: (0, 0)),
                pl.BlockSpec((FF, H), lambda tb: (0, 0)),
                pl.BlockSpec((FF, H), lambda tb: (0, 0)),
                pl.BlockSpec((H, FF), lambda tb: (0, 0)),
            ],
            out_specs=pl.BlockSpec((TM, H), lambda tb: (tb, 0)),
        ),
        out_shape=jax.ShapeDtypeStruct((T, H), f32),
        compiler_params=pltpu.CompilerParams(
            dimension_semantics=("arbitrary",)),
    )(x_bf, pw, ys_bf, wg_bf, wu_bf, wd_bf)

    return out


# speedup vs baseline: 1.6508x; 1.6508x over previous
import functools

import jax
import jax.numpy as jnp
from jax.experimental import pallas as pl
from jax.experimental.pallas import tpu as pltpu

H = 1024
FF = 2048
E = 8
T = 512
TM = 128
NT = 16
NS = NT * TM
TB = T // TM


def _router_kernel(x_ref, rw_ref, pw_ref, counts_ref):
    x = x_ref[:, :]
    rw = rw_ref[:, :]
    logits = jax.lax.dot_general(
        x, rw, (((1,), (1,)), ((), ())), preferred_element_type=jnp.float32)
    m = jnp.max(logits, axis=1, keepdims=True)
    ex = jnp.exp(logits - m)
    probs = ex / jnp.sum(ex, axis=1, keepdims=True)

    ecols = jax.lax.broadcasted_iota(jnp.int32, (T, E), 1)
    m1 = jnp.max(probs, axis=1, keepdims=True)
    i1 = jnp.min(jnp.where(probs == m1, ecols, E), axis=1, keepdims=True)
    probs2 = jnp.where(ecols == i1, -jnp.inf, probs)
    m2 = jnp.max(probs2, axis=1, keepdims=True)
    i2 = jnp.min(jnp.where(probs2 == m2, ecols, E), axis=1, keepdims=True)
    s = m1 + m2
    w1 = m1 / s
    w2 = m2 / s

    sel1 = (ecols == i1)
    sel2 = (ecols == i2)
    memb = (sel1 | sel2).astype(jnp.float32)

    ri = jax.lax.broadcasted_iota(jnp.int32, (T, T), 0)
    ci = jax.lax.broadcasted_iota(jnp.int32, (T, T), 1)
    ltri = (ci < ri).astype(jnp.float32)
    rank = jax.lax.dot_general(
        ltri, memb, (((1,), (0,)), ((), ())), preferred_element_type=jnp.float32)

    counts = jnp.sum(memb, axis=0, keepdims=True)
    tiles = jnp.ceil(counts / TM)
    er = jax.lax.broadcasted_iota(jnp.int32, (E, E), 0)
    ec = jax.lax.broadcasted_iota(jnp.int32, (E, E), 1)
    ltri_e = (er < ec).astype(jnp.float32)
    base = jax.lax.dot_general(
        tiles, ltri_e, (((1,), (0,)), ((), ())),
        preferred_element_type=jnp.float32) * TM

    pos = base + rank
    p1 = jnp.sum(jnp.where(sel1, pos, 0.0), axis=1, keepdims=True)
    p2 = jnp.sum(jnp.where(sel2, pos, 0.0), axis=1, keepdims=True)

    pw_ref[:, :] = (jnp.where(ecols == 0, p1, 0.0) +
                    jnp.where(ecols == 1, p2, 0.0) +
                    jnp.where(ecols == 2, w1, 0.0) +
                    jnp.where(ecols == 3, w2, 0.0))
    counts_ref[:, :] = jnp.broadcast_to(counts, (E, E))


def _ffn_kernel(te_ref, ta_ref, x_ref, pt_ref, eg_ref, eu_ref, ed_ref, ys_ref):
    nt = pl.program_id(0)

    @pl.when(ta_ref[nt] == 1)
    def _active():
        si = (jax.lax.broadcasted_iota(jnp.int32, (TM, T), 0)
              + nt * TM).astype(jnp.float32)
        d = ((pt_ref[0:1, :] == si) | (pt_ref[1:2, :] == si)).astype(jnp.float32)
        xs = jax.lax.dot_general(
            d, x_ref[:, :], (((1,), (0,)), ((), ())),
            preferred_element_type=jnp.float32)
        xg = jax.lax.dot_general(
            xs, eg_ref[0], (((1,), (1,)), ((), ())),
            preferred_element_type=jnp.float32)
        xu = jax.lax.dot_general(
            xs, eu_ref[0], (((1,), (1,)), ((), ())),
            preferred_element_type=jnp.float32)
        h = xg * jax.nn.sigmoid(xg) * xu
        ys_ref[:, :] = jax.lax.dot_general(
            h, ed_ref[0], (((1,), (1,)), ((), ())),
            preferred_element_type=jnp.float32)

    @pl.when(ta_ref[nt] == 0)
    def _inactive():
        ys_ref[:, :] = jnp.zeros((TM, H), jnp.float32)


def _combine_kernel(x_ref, pw_ref, ys_ref, wg_ref, wu_ref, wd_ref, o_ref):
    xb = x_ref[:, :]
    xg = jax.lax.dot_general(
        xb, wg_ref[:, :], (((1,), (1,)), ((), ())),
        preferred_element_type=jnp.float32)
    xu = jax.lax.dot_general(
        xb, wu_ref[:, :], (((1,), (1,)), ((), ())),
        preferred_element_type=jnp.float32)
    h = xg * jax.nn.sigmoid(xg) * xu
    shared = jax.lax.dot_general(
        h, wd_ref[:, :], (((1,), (1,)), ((), ())),
        preferred_element_type=jnp.float32)

    p1 = pw_ref[:, 0:1]
    p2 = pw_ref[:, 1:2]
    w1 = pw_ref[:, 2:3]
    w2 = pw_ref[:, 3:4]
    sj = jax.lax.broadcasted_iota(jnp.int32, (TM, NS), 1).astype(jnp.float32)
    c = (w1 * (p1 == sj).astype(jnp.float32) +
         w2 * (p2 == sj).astype(jnp.float32))
    moe = jax.lax.dot_general(
        c, ys_ref[:, :], (((1,), (0,)), ((), ())),
        preferred_element_type=jnp.float32)
    o_ref[:, :] = shared + moe


@jax.jit
def kernel(x, router_weight, w_gate, w_up, w_down,
           expert_gate, expert_up, expert_down):
    f32 = jnp.float32

    pw, counts8 = pl.pallas_call(
        _router_kernel,
        out_shape=(jax.ShapeDtypeStruct((T, E), f32),
                   jax.ShapeDtypeStruct((E, E), f32)),
    )(x, router_weight)

    counts = counts8[0].astype(jnp.int32)
    tiles = (counts + TM - 1) // TM
    ends = jnp.cumsum(tiles)
    total = ends[E - 1]
    ntv = jnp.arange(NT, dtype=jnp.int32)
    ntc = jnp.minimum(ntv, total - 1)
    tile_expert = jnp.sum((ends[None, :] <= ntc[:, None]).astype(jnp.int32),
                          axis=1)
    tile_active = (ntv < total).astype(jnp.int32)

    pt = jnp.full((8, T), -1.0, f32)
    pt = pt.at[0].set(pw[:, 0]).at[1].set(pw[:, 1])

    ys = pl.pallas_call(
        _ffn_kernel,
        grid_spec=pltpu.PrefetchScalarGridSpec(
            num_scalar_prefetch=2,
            grid=(NT,),
            in_specs=[
                pl.BlockSpec((T, H), lambda nt, te, ta: (0, 0)),
                pl.BlockSpec((8, T), lambda nt, te, ta: (0, 0)),
                pl.BlockSpec((1, FF, H), lambda nt, te, ta: (te[nt], 0, 0)),
                pl.BlockSpec((1, FF, H), lambda nt, te, ta: (te[nt], 0, 0)),
                pl.BlockSpec((1, H, FF), lambda nt, te, ta: (te[nt], 0, 0)),
            ],
            out_specs=pl.BlockSpec((TM, H), lambda nt, te, ta: (nt, 0)),
        ),
        out_shape=jax.ShapeDtypeStruct((NS, H), f32),
        compiler_params=pltpu.CompilerParams(
            dimension_semantics=("arbitrary",)),
    )(tile_expert, tile_active, x, pt, expert_gate, expert_up, expert_down)

    out = pl.pallas_call(
        _combine_kernel,
        grid_spec=pl.GridSpec(
            grid=(TB,),
            in_specs=[
                pl.BlockSpec((TM, H), lambda tb: (tb, 0)),
                pl.BlockSpec((TM, E), lambda tb: (tb, 0)),
                pl.BlockSpec((NS, H), lambda tb: (0, 0)),
                pl.BlockSpec((FF, H), lambda tb: (0, 0)),
                pl.BlockSpec((FF, H), lambda tb: (0, 0)),
                pl.BlockSpec((H, FF), lambda tb: (0, 0)),
            ],
            out_specs=pl.BlockSpec((TM, H), lambda tb: (tb, 0)),
        ),
        out_shape=jax.ShapeDtypeStruct((T, H), f32),
        compiler_params=pltpu.CompilerParams(
            dimension_semantics=("arbitrary",)),
    )(x, pw, ys, w_gate, w_up, w_down)

    return out


# speedup vs baseline: 1.7358x; 1.0515x over previous
import jax
import jax.numpy as jnp
from jax.experimental import pallas as pl
from jax.experimental.pallas import tpu as pltpu

H = 1024
FF = 2048
E = 8
T = 512
TM = 128
NTE = 4
NS = 2048
FFB = 1024
NFB = FF // FFB
TB = T // TM
BF = jnp.bfloat16


def _router_kernel(x_ref, rw_ref, pw_ref, counts_ref):
    x = x_ref[:, :]
    rw = rw_ref[:, :]
    logits = jax.lax.dot_general(
        x, rw, (((1,), (1,)), ((), ())), preferred_element_type=jnp.float32)
    m = jnp.max(logits, axis=1, keepdims=True)
    ex = jnp.exp(logits - m)
    probs = ex / jnp.sum(ex, axis=1, keepdims=True)

    ecols = jax.lax.broadcasted_iota(jnp.int32, (T, E), 1)
    m1 = jnp.max(probs, axis=1, keepdims=True)
    i1 = jnp.min(jnp.where(probs == m1, ecols, E), axis=1, keepdims=True)
    probs2 = jnp.where(ecols == i1, -jnp.inf, probs)
    m2 = jnp.max(probs2, axis=1, keepdims=True)
    i2 = jnp.min(jnp.where(probs2 == m2, ecols, E), axis=1, keepdims=True)
    s = m1 + m2
    w1 = m1 / s
    w2 = m2 / s

    sel1 = (ecols == i1)
    sel2 = (ecols == i2)
    memb = (sel1 | sel2).astype(jnp.float32)

    ri = jax.lax.broadcasted_iota(jnp.int32, (T, T), 0)
    ci = jax.lax.broadcasted_iota(jnp.int32, (T, T), 1)
    ltri = (ci < ri).astype(jnp.float32)
    rank = jax.lax.dot_general(
        ltri, memb, (((1,), (0,)), ((), ())), preferred_element_type=jnp.float32)

    counts = jnp.sum(memb, axis=0, keepdims=True)
    tiles = jnp.ceil(counts / TM)
    er = jax.lax.broadcasted_iota(jnp.int32, (E, E), 0)
    ec = jax.lax.broadcasted_iota(jnp.int32, (E, E), 1)
    ltri_e = (er < ec).astype(jnp.float32)
    base = jax.lax.dot_general(
        tiles, ltri_e, (((1,), (0,)), ((), ())),
        preferred_element_type=jnp.float32) * TM

    pos = base + rank
    p1 = jnp.sum(jnp.where(sel1, pos, 0.0), axis=1, keepdims=True)
    p2 = jnp.sum(jnp.where(sel2, pos, 0.0), axis=1, keepdims=True)

    pw_ref[:, :] = (jnp.where(ecols == 0, p1, 0.0) +
                    jnp.where(ecols == 1, p2, 0.0) +
                    jnp.where(ecols == 2, w1, 0.0) +
                    jnp.where(ecols == 3, w2, 0.0))
    counts_ref[:, :] = jnp.broadcast_to(counts, (E, E))


def _ffn_kernel(eb_ref, en_ref, x_ref, pt_ref, eg_ref, eu_ref, ed_ref,
                ys_ref, xbf_ref, acc_ref):
    e = pl.program_id(0)
    fb = pl.program_id(1)

    @pl.when(jnp.logical_and(e == 0, fb == 0))
    def _init():
        ys_ref[:, :] = jnp.zeros((NS, H), BF)
        xbf_ref[:, :] = x_ref[:, :].astype(BF)

    gate_bf = eg_ref[0].astype(BF)
    up_bf = eu_ref[0].astype(BF)
    down_bf = ed_ref[0].astype(BF)
    base = eb_ref[e]
    ntile = en_ref[e]

    for i in range(NTE):
        @pl.when(i < ntile)
        def _tile(i=i):
            row0 = pl.multiple_of(base + i * TM, TM)
            si = (jax.lax.broadcasted_iota(jnp.int32, (TM, T), 0)
                  + row0).astype(jnp.float32)
            d = ((pt_ref[0:1, :] == si) |
                 (pt_ref[1:2, :] == si)).astype(BF)
            xs = jax.lax.dot_general(
                d, xbf_ref[:, :], (((1,), (0,)), ((), ())),
                preferred_element_type=jnp.float32).astype(BF)
            xg = jax.lax.dot_general(
                xs, gate_bf, (((1,), (1,)), ((), ())),
                preferred_element_type=jnp.float32)
            xu = jax.lax.dot_general(
                xs, up_bf, (((1,), (1,)), ((), ())),
                preferred_element_type=jnp.float32)
            h = (xg * jax.nn.sigmoid(xg) * xu).astype(BF)
            part = jax.lax.dot_general(
                h, down_bf, (((1,), (1,)), ((), ())),
                preferred_element_type=jnp.float32)

            @pl.when(fb == 0)
            def _first():
                acc_ref[pl.ds(row0, TM), :] = part

            @pl.when(fb == 1)
            def _last():
                ys_ref[pl.ds(row0, TM), :] = (
                    acc_ref[pl.ds(row0, TM), :] + part).astype(BF)


def _combine_kernel(x_ref, pw_ref, ys_ref, wg_ref, wu_ref, wd_ref, o_ref,
                    wgb_ref, wub_ref, wdb_ref):
    tb = pl.program_id(0)

    @pl.when(tb == 0)
    def _precast():
        wgb_ref[:, :] = wg_ref[:, :].astype(BF)
        wub_ref[:, :] = wu_ref[:, :].astype(BF)
        wdb_ref[:, :] = wd_ref[:, :].astype(BF)

    xb = x_ref[:, :].astype(BF)
    xg = jax.lax.dot_general(
        xb, wgb_ref[:, :], (((1,), (1,)), ((), ())),
        preferred_element_type=jnp.float32)
    xu = jax.lax.dot_general(
        xb, wub_ref[:, :], (((1,), (1,)), ((), ())),
        preferred_element_type=jnp.float32)
    h = (xg * jax.nn.sigmoid(xg) * xu).astype(BF)
    shared = jax.lax.dot_general(
        h, wdb_ref[:, :], (((1,), (1,)), ((), ())),
        preferred_element_type=jnp.float32)

    p1 = pw_ref[:, 0:1]
    p2 = pw_ref[:, 1:2]
    w1 = pw_ref[:, 2:3]
    w2 = pw_ref[:, 3:4]
    sj = jax.lax.broadcasted_iota(jnp.int32, (TM, NS), 1).astype(jnp.float32)
    c = (w1 * (p1 == sj).astype(jnp.float32) +
         w2 * (p2 == sj).astype(jnp.float32)).astype(BF)
    moe = jax.lax.dot_general(
        c, ys_ref[:, :], (((1,), (0,)), ((), ())),
        preferred_element_type=jnp.float32)
    o_ref[:, :] = shared + moe


@jax.jit
def kernel(x, router_weight, w_gate, w_up, w_down,
           expert_gate, expert_up, expert_down):
    f32 = jnp.float32

    pw, counts8 = pl.pallas_call(
        _router_kernel,
        out_shape=(jax.ShapeDtypeStruct((T, E), f32),
                   jax.ShapeDtypeStruct((E, E), f32)),
    )(x, router_weight)

    counts = counts8[0].astype(jnp.int32)
    tiles = (counts + TM - 1) // TM
    base_rows = (jnp.cumsum(tiles) - tiles) * TM

    pt = jnp.full((8, T), -1.0, f32)
    pt = pt.at[0].set(pw[:, 0]).at[1].set(pw[:, 1])

    ys = pl.pallas_call(
        _ffn_kernel,
        grid_spec=pltpu.PrefetchScalarGridSpec(
            num_scalar_prefetch=2,
            grid=(E, NFB),
            in_specs=[
                pl.BlockSpec((T, H), lambda e, fb, eb, en: (0, 0)),
                pl.BlockSpec((8, T), lambda e, fb, eb, en: (0, 0)),
                pl.BlockSpec((1, FFB, H), lambda e, fb, eb, en: (e, fb, 0)),
                pl.BlockSpec((1, FFB, H), lambda e, fb, eb, en: (e, fb, 0)),
                pl.BlockSpec((1, H, FFB), lambda e, fb, eb, en: (e, 0, fb)),
            ],
            out_specs=pl.BlockSpec((NS, H), lambda e, fb, eb, en: (0, 0)),
            scratch_shapes=[
                pltpu.VMEM((T, H), BF),
                pltpu.VMEM((NS, H), jnp.float32),
            ],
        ),
        out_shape=jax.ShapeDtypeStruct((NS, H), BF),
        compiler_params=pltpu.CompilerParams(
            dimension_semantics=("arbitrary", "arbitrary")),
    )(base_rows, tiles, x, pt, expert_gate, expert_up, expert_down)

    out = pl.pallas_call(
        _combine_kernel,
        grid=(TB,),
        in_specs=[
            pl.BlockSpec((TM, H), lambda tb: (tb, 0)),
            pl.BlockSpec((TM, E), lambda tb: (tb, 0)),
            pl.BlockSpec((NS, H), lambda tb: (0, 0)),
            pl.BlockSpec((FF, H), lambda tb: (0, 0)),
            pl.BlockSpec((FF, H), lambda tb: (0, 0)),
            pl.BlockSpec((H, FF), lambda tb: (0, 0)),
        ],
        out_specs=pl.BlockSpec((TM, H), lambda tb: (tb, 0)),
        out_shape=jax.ShapeDtypeStruct((T, H), f32),
        scratch_shapes=[
            pltpu.VMEM((FF, H), BF),
            pltpu.VMEM((FF, H), BF),
            pltpu.VMEM((H, FF), BF),
        ],
        compiler_params=pltpu.CompilerParams(
            dimension_semantics=("arbitrary",)),
    )(x, pw, ys, w_gate, w_up, w_down)

    return out
